# baseline (speedup 1.0000x reference)
import functools

import jax
import jax.numpy as jnp
from jax import lax
from jax.experimental import pallas as pl
from jax.experimental.pallas import tpu as pltpu
from jax.experimental.pallas import tpu_sc as plsc

V = 8
S = 7
D = 64
B = 16384

NC = 2
NS = 16
L = 16
NW = NC * NS
BPW = B // NW
CHUNK = 128
NCH = BPW // CHUNK
GPC = CHUNK // L
SR = 2 * D

_MESH = plsc.VectorSubcoreMesh(core_axis_name="c", subcore_axis_name="s")
_CP_TILED = pltpu.CompilerParams(use_tc_tiling_on_sc=True,
                                 needs_layout_passes=False)
_CP_LINEAR = pltpu.CompilerParams(use_tc_tiling_on_sc=False,
                                  needs_layout_passes=False)


def _worker_base():
    wid = lax.axis_index("s") * NC + lax.axis_index("c")
    return wid * BPW


def _zero_rows(buf, nrows, ncols):
    def body(r, _):
        for c in range(ncols // L):
            buf[r, pl.ds(c * L, L)] = jnp.zeros((L,), jnp.float32)
        return _
    lax.fori_loop(0, nrows, body, None)


def _start_gathers(table, idx_row, idx_v, gbuf, sem):
    copies = [pltpu.make_async_copy(table.at[idx_v.at[idx_row, j]],
                                    gbuf.at[pl.ds(j * CHUNK, CHUNK)], sem)
              for j in range(NCH)]
    for c in copies:
        c.start()
    return copies


def _start_block_writes(obuf, out_ref, rows0, base, wsem):
    writes = [pltpu.make_async_copy(
        obuf.at[pl.ds(dg * 8, 8)],
        out_ref.at[pl.ds(rows0 + dg * 8, 8), pl.ds(base, BPW)], wsem)
        for dg in range(D // 8)]
    for w in writes:
        w.start()
    return writes


def _drain(copies):
    for c in copies:
        c.wait()


def _k1_body(inT, t1, t2, t3, t4, e56_out, out_ref,
             in_v, idx_v, pbuf, gbuf, obuf, ebuf, sem, wsem):
    base = _worker_base()

    pltpu.sync_copy(inT.at[pl.ds(0, 6), :, pl.ds(base, BPW)], in_v)

    _zero_rows(obuf, D, BPW)
    pending = _start_block_writes(obuf, out_ref, 0, base, wsem)
    _zero_rows(ebuf, 8, BPW)

    for j in range(NCH):
        def amax(g, _):
            off = j * CHUNK + g * L
            e = jnp.zeros((L,), jnp.int32)
            for s in range(S - 1):
                m = in_v[s, 0, pl.ds(off, L)]
                a = jnp.zeros((L,), jnp.int32)
                for v in range(1, V):
                    val = in_v[s, v, pl.ds(off, L)]
                    gt = val > m
                    m = jnp.where(gt, val, m)
                    a = jnp.where(gt, jnp.full((L,), v, jnp.int32), a)
                if s == 0:
                    pbuf[pl.ds(off, L)] = (a & 1) * D
                e = e + a * (V ** s)
                if s < 4:
                    idx_v[s, j, pl.ds(g * L, L)] = e >> 1
                else:
                    ebuf[s - 4, pl.ds(off, L)] = plsc.bitcast(e, jnp.float32)
            return _
        lax.fori_loop(0, GPC, amax, None)

    ew = pltpu.make_async_copy(ebuf, e56_out.at[:, pl.ds(base, BPW)], wsem)
    ew.start()

    iota = lax.iota(jnp.int32, L)
    tabs = [t1, t2, t3, t4]
    gathers = _start_gathers(tabs[0], 0, idx_v, gbuf, sem)
    for t in range(4):
        _drain(gathers)
        _drain(pending)

        def tcol(i, _):
            row = i * L + iota
            pv = pbuf[pl.ds(i * L, L)]
            for d in range(D):
                v = plsc.load_gather(gbuf, [row, pv + d])
                obuf[d, pl.ds(i * L, L)] = v
            return _
        lax.fori_loop(0, BPW // L, tcol, None)
        if t < 3:
            gathers = _start_gathers(tabs[t + 1], t + 1, idx_v, gbuf, sem)
        pending = _start_block_writes(obuf, out_ref, (t + 1) * D, base, wsem)
    _drain(pending)
    ew.wait()


def _k2_body(t5, t6, e56, out2, ebuf, idx_v, gbuf, obuf, sem, wsem):
    base = _worker_base()
    pltpu.sync_copy(e56.at[:, pl.ds(base, BPW)], ebuf)

    for j in range(NCH):
        for g in range(GPC):
            off = j * CHUNK + g * L
            e5 = plsc.bitcast(ebuf[0, pl.ds(off, L)], jnp.int32)
            e6 = plsc.bitcast(ebuf[1, pl.ds(off, L)], jnp.int32)
            idx_v[0, j, pl.ds(g * L, L)] = e5
            idx_v[1, j, pl.ds(g * L, L)] = e6

    iota = lax.iota(jnp.int32, L)
    pending = []
    gathers = _start_gathers(t5, 0, idx_v, gbuf, sem)
    for t, tab in enumerate([t5, t6]):
        _drain(gathers)

        def tcol(i, _):
            row = i * L + iota
            for d in range(D):
                v = plsc.load_gather(gbuf, [row, jnp.full((L,), d,
                                                          jnp.int32)])
                obuf[t, d, pl.ds(i * L, L)] = v
            return _
        lax.fori_loop(0, BPW // L, tcol, None)
        if t == 0:
            gathers = _start_gathers(t6, 1, idx_v, gbuf, sem)
        w = pltpu.make_async_copy(
            obuf.at[t], out2.at[pl.ds(t * D, D), pl.ds(base, BPW)], wsem)
        w.start()
        pending.append(w)
    _drain(pending)


def _k3_body(out2t, out_ref, vbuf, sem):
    base = _worker_base()
    pltpu.sync_copy(out2t.at[:, pl.ds(base, BPW)], vbuf)
    pltpu.sync_copy(vbuf, out_ref.at[pl.ds(5 * D, 2 * D), pl.ds(base, BPW)])


_k1 = functools.partial(
    pl.kernel,
    out_type=(jax.ShapeDtypeStruct((8, B), jnp.float32),
              jax.ShapeDtypeStruct((S * D, B), jnp.float32)),
    mesh=_MESH,
    compiler_params=_CP_TILED,
    scratch_types=[
        pltpu.VMEM((6, V, BPW), jnp.float32),
        pltpu.VMEM((4, NCH, CHUNK), jnp.int32),
        pltpu.VMEM((BPW,), jnp.int32),
        pltpu.VMEM((BPW, SR), jnp.float32),
        pltpu.VMEM((D, BPW), jnp.float32),
        pltpu.VMEM((8, BPW), jnp.float32),
        pltpu.SemaphoreType.DMA,
        pltpu.SemaphoreType.DMA,
    ],
)(_k1_body)

_k2 = functools.partial(
    pl.kernel,
    out_type=jax.ShapeDtypeStruct((2 * D, B), jnp.float32),
    mesh=_MESH,
    compiler_params=_CP_LINEAR,
    scratch_types=[
        pltpu.VMEM((8, BPW), jnp.float32),
        pltpu.VMEM((2, NCH, CHUNK), jnp.int32),
        pltpu.VMEM((BPW, D), jnp.float32),
        pltpu.VMEM((2, D, BPW), jnp.float32),
        pltpu.SemaphoreType.DMA,
        pltpu.SemaphoreType.DMA,
    ],
)(_k2_body)

_k3 = functools.partial(
    pl.kernel,
    out_type=(),
    mesh=_MESH,
    compiler_params=_CP_TILED,
    scratch_types=[
        pltpu.VMEM((2 * D, BPW), jnp.float32),
        pltpu.SemaphoreType.DMA,
    ],
)(_k3_body)


@jax.jit
def _run(inputs, t1, t2, t3, t4, t5, t6):
    inT = jnp.transpose(inputs, (1, 2, 0))
    e56, out1 = _k1(inT, t1.reshape(-1, SR), t2.reshape(-1, SR),
                    t3.reshape(-1, SR), t4.reshape(-1, SR))
    out2 = _k2(t5, t6, e56)
    out_ref = jax.new_ref(out1)
    _k3(out2, out_ref)
    out = out_ref[...]
    return jnp.transpose(out, (1, 0)).reshape(B, S, D)


def kernel(inputs, table_1, table_2, table_3, table_4, table_5, table_6):
    return _run(inputs, table_1, table_2, table_3, table_4,
                table_5, table_6)


# speedup vs baseline: 1.4379x; 1.4379x over previous
import functools

import jax
import jax.numpy as jnp
from jax import lax
from jax.experimental import pallas as pl
from jax.experimental.pallas import tpu as pltpu
from jax.experimental.pallas import tpu_sc as plsc

V = 8
S = 7
D = 64
B = 16384

NC = 2
NS = 16
L = 16
NW = NC * NS
BPW = B // NW
CHUNK = 128
NCH = BPW // CHUNK
GPC = CHUNK // L

_MESH = plsc.VectorSubcoreMesh(core_axis_name="c", subcore_axis_name="s")
_CP = pltpu.CompilerParams(use_tc_tiling_on_sc=False,
                           needs_layout_passes=False)


def _body(inT, t1, t2, t3, t4, t5, t6, out_ref,
          in_v, idx_v, zbuf, gbuf, sem, wsem):
    wid = lax.axis_index("s") * NC + lax.axis_index("c")
    base = wid * BPW
    tabs = [t1, t2, t3, t4, t5, t6]

    def zrow(r, _):
        for c in range(D // L):
            zbuf[r, pl.ds(c * L, L)] = jnp.zeros((L,), jnp.float32)
        return _
    lax.fori_loop(0, CHUNK, zrow, None)

    for j in range(NCH):
        pltpu.sync_copy(inT.at[pl.ds(0, 6), :,
                               pl.ds(base + j * CHUNK, CHUNK)], in_v)

        def amax(g, _):
            e = jnp.zeros((L,), jnp.int32)
            for s in range(S - 1):
                m = in_v[s, 0, pl.ds(g * L, L)]
                a = jnp.zeros((L,), jnp.int32)
                for v in range(1, V):
                    val = in_v[s, v, pl.ds(g * L, L)]
                    gt = val > m
                    m = jnp.where(gt, val, m)
                    a = jnp.where(gt, jnp.full((L,), v, jnp.int32), a)
                e = e + a * (V ** s)
                idx_v[s, j, pl.ds(g * L, L)] = e
            return _
        lax.fori_loop(0, GPC, amax, None)

    def start_gathers(j, bank):
        copies = [pltpu.make_async_copy(
            tabs[t].at[idx_v.at[t, j]], gbuf.at[bank, t], sem)
            for t in range(6)]
        for c in copies:
            c.start()
        return copies

    def start_writes(j, bank):
        rbase = base + j * CHUNK
        writes = [pltpu.make_async_copy(
            zbuf, out_ref.at[pl.ds(rbase, CHUNK), pl.ds(0, D)], wsem)]
        for t in range(6):
            writes.append(pltpu.make_async_copy(
                gbuf.at[bank, t],
                out_ref.at[pl.ds(rbase, CHUNK), pl.ds((t + 1) * D, D)],
                wsem))
        for w in writes:
            w.start()
        return writes

    gathers = start_gathers(0, 0)
    pending = []
    for j in range(NCH):
        for c in gathers:
            c.wait()
        if j >= 1:
            for w in pending[j - 1]:
                w.wait()
        if j < NCH - 1:
            gathers = start_gathers(j + 1, (j + 1) % 2)
        pending.append(start_writes(j, j % 2))
    for w in pending[NCH - 1]:
        w.wait()


_kern = functools.partial(
    pl.kernel,
    out_type=jax.ShapeDtypeStruct((B, S * D), jnp.float32),
    mesh=_MESH,
    compiler_params=_CP,
    scratch_types=[
        pltpu.VMEM((6, V, CHUNK), jnp.float32),
        pltpu.VMEM((6, NCH, CHUNK), jnp.int32),
        pltpu.VMEM((CHUNK, D), jnp.float32),
        pltpu.VMEM((2, 6, CHUNK, D), jnp.float32),
        pltpu.SemaphoreType.DMA,
        pltpu.SemaphoreType.DMA,
    ],
)(_body)


@jax.jit
def _run(inputs, t1, t2, t3, t4, t5, t6):
    inT = jnp.transpose(inputs, (1, 2, 0))
    out = _kern(inT, t1, t2, t3, t4, t5, t6)
    return out.reshape(B, S, D)


def kernel(inputs, table_1, table_2, table_3, table_4, table_5, table_6):
    return _run(inputs, table_1, table_2, table_3, table_4,
                table_5, table_6)


# speedup vs baseline: 1.4455x; 1.0053x over previous
import functools

import jax
import jax.numpy as jnp
from jax import lax
from jax.experimental import pallas as pl
from jax.experimental.pallas import tpu as pltpu
from jax.experimental.pallas import tpu_sc as plsc

V = 8
S = 7
D = 64
B = 16384

NC = 2
NS = 16
L = 16
NW = NC * NS
BPW = B // NW
HALF = BPW // 2
CHUNK = 128
NCH = BPW // CHUNK
GPC = CHUNK // L

_MESH = plsc.VectorSubcoreMesh(core_axis_name="c", subcore_axis_name="s")
_CP = pltpu.CompilerParams(use_tc_tiling_on_sc=False,
                           needs_layout_passes=False)


def _body(in_hbm, t1, t2, t3, t4, t5, t6, out_ref,
          in_v, idx_v, zbuf, gbuf, sem, wsem):
    wid = lax.axis_index("s") * NC + lax.axis_index("c")
    base = wid * BPW
    tabs = [t1, t2, t3, t4, t5, t6]
    iota = lax.iota(jnp.int32, L)

    def zrow(r, _):
        for c in range(D // L):
            zbuf[r, pl.ds(c * L, L)] = jnp.zeros((L,), jnp.float32)
        return _
    lax.fori_loop(0, CHUNK, zrow, None)

    for h in range(2):
        pltpu.sync_copy(in_hbm.at[pl.ds(base + h * HALF, HALF)], in_v)
        for j in range(NCH // 2):
            def amax(g, _):
                row = j * CHUNK + g * L + iota
                e = jnp.zeros((L,), jnp.int32)
                for s in range(S - 1):
                    col0 = jnp.full((L,), s * V, jnp.int32)
                    m = plsc.load_gather(in_v, [row, col0])
                    a = jnp.zeros((L,), jnp.int32)
                    for v in range(1, V):
                        colv = jnp.full((L,), s * V + v, jnp.int32)
                        val = plsc.load_gather(in_v, [row, colv])
                        gt = val > m
                        m = jnp.where(gt, val, m)
                        a = jnp.where(gt, jnp.full((L,), v, jnp.int32), a)
                    e = e + a * (V ** s)
                    idx_v[s, h * 2 + j, pl.ds(g * L, L)] = e
                return _
            lax.fori_loop(0, GPC, amax, None)

    def start_gathers(j, bank):
        copies = [pltpu.make_async_copy(
            tabs[t].at[idx_v.at[t, j]], gbuf.at[bank, t], sem)
            for t in range(6)]
        for c in copies:
            c.start()
        return copies

    def start_writes(j, bank):
        rbase = base + j * CHUNK
        writes = [pltpu.make_async_copy(
            zbuf, out_ref.at[pl.ds(rbase, CHUNK), pl.ds(0, D)], wsem)]
        for t in range(6):
            writes.append(pltpu.make_async_copy(
                gbuf.at[bank, t],
                out_ref.at[pl.ds(rbase, CHUNK), pl.ds((t + 1) * D, D)],
                wsem))
        for w in writes:
            w.start()
        return writes

    gathers = start_gathers(0, 0)
    pending = []
    for j in range(NCH):
        for c in gathers:
            c.wait()
        if j >= 1:
            for w in pending[j - 1]:
                w.wait()
        if j < NCH - 1:
            gathers = start_gathers(j + 1, (j + 1) % 2)
        pending.append(start_writes(j, j % 2))
    for w in pending[NCH - 1]:
        w.wait()


_kern = functools.partial(
    pl.kernel,
    out_type=jax.ShapeDtypeStruct((B, S * D), jnp.float32),
    mesh=_MESH,
    compiler_params=_CP,
    scratch_types=[
        pltpu.VMEM((HALF, S * V), jnp.float32),
        pltpu.VMEM((6, NCH, CHUNK), jnp.int32),
        pltpu.VMEM((CHUNK, D), jnp.float32),
        pltpu.VMEM((2, 6, CHUNK, D), jnp.float32),
        pltpu.SemaphoreType.DMA,
        pltpu.SemaphoreType.DMA,
    ],
)(_body)


@jax.jit
def _run(inputs2d, t1, t2, t3, t4, t5, t6):
    out = _kern(inputs2d, t1, t2, t3, t4, t5, t6)
    return out.reshape(B, S, D)


def kernel(inputs, table_1, table_2, table_3, table_4, table_5, table_6):
    return _run(inputs.reshape(B, S * V), table_1, table_2, table_3,
                table_4, table_5, table_6)


# speedup vs baseline: 1.4876x; 1.0291x over previous
import functools

import jax
import jax.numpy as jnp
from jax import lax
from jax.experimental import pallas as pl
from jax.experimental.pallas import tpu as pltpu
from jax.experimental.pallas import tpu_sc as plsc

V = 8
S = 7
D = 64
B = 16384

NC = 2
NS = 16
L = 16
NW = NC * NS
BPW = B // NW
CHUNK = 128
NCH = BPW // CHUNK
GPC = CHUNK // L


def _body(in_hbm, t1, t2, t3, t4, t5, t6, out_ref,
          in_v, idx_v, gbuf, zbuf, sem, wsem):
    wid = lax.axis_index("s") * NC + lax.axis_index("c")
    base = wid * BPW
    tabs = [t1, t2, t3, t4, t5, t6]

    pltpu.sync_copy(in_hbm.at[pl.ds(base, BPW)], in_v)

    iota = jax.lax.iota(jnp.int32, L)

    def zero_body(r, _):
        for c in range(D // L):
            zbuf[r, pl.ds(c * L, L)] = jnp.zeros((L,), jnp.float32)
        return _
    lax.fori_loop(0, CHUNK, zero_body, None)

    for j in range(NCH):
        def amax_body(gg, _):
            row = j * CHUNK + gg * L + iota
            e = jnp.zeros((L,), jnp.int32)
            for s in range(S - 1):
                col0 = jnp.full((L,), s * V, jnp.int32)
                m = plsc.load_gather(in_v, [row, col0])
                a = jnp.zeros((L,), jnp.int32)
                for v in range(1, V):
                    colv = jnp.full((L,), s * V + v, jnp.int32)
                    val = plsc.load_gather(in_v, [row, colv])
                    gt = val > m
                    m = jnp.where(gt, val, m)
                    a = jnp.where(gt, jnp.full((L,), v, jnp.int32), a)
                e = e + a * (V ** s)
                idx_v[s, j, pl.ds(gg * L, L)] = e
            return _
        lax.fori_loop(0, GPC, amax_body, None)

        copies = []
        for d in range(6):
            copies.append(pltpu.make_async_copy(
                tabs[d].at[idx_v.at[d, j]], gbuf.at[d], sem))
        for c in copies:
            c.start()
        for c in copies:
            c.wait()

        rbase = base + j * CHUNK
        writes = [pltpu.make_async_copy(
            zbuf, out_ref.at[pl.ds(rbase, CHUNK), pl.ds(0, D)], wsem)]
        for d in range(6):
            writes.append(pltpu.make_async_copy(
                gbuf.at[d],
                out_ref.at[pl.ds(rbase, CHUNK), pl.ds((d + 1) * D, D)],
                wsem))
        for w in writes:
            w.start()
        for w in writes:
            w.wait()


_kern = functools.partial(
    pl.kernel,
    out_type=jax.ShapeDtypeStruct((B, S * D), jnp.float32),
    mesh=plsc.VectorSubcoreMesh(core_axis_name="c", subcore_axis_name="s"),
    compiler_params=pltpu.CompilerParams(use_tc_tiling_on_sc=False,
                                         needs_layout_passes=False),
    scratch_types=[
        pltpu.VMEM((BPW, S * V), jnp.float32),
        pltpu.VMEM((6, NCH, CHUNK), jnp.int32),
        pltpu.VMEM((6, CHUNK, D), jnp.float32),
        pltpu.VMEM((CHUNK, D), jnp.float32),
        pltpu.SemaphoreType.DMA,
        pltpu.SemaphoreType.DMA,
    ],
)(_body)


@jax.jit
def _run(inputs2d, t1, t2, t3, t4, t5, t6):
    return _kern(inputs2d, t1, t2, t3, t4, t5, t6).reshape(B, S, D)


def kernel(inputs, table_1, table_2, table_3, table_4, table_5, table_6):
    return _run(inputs.reshape(B, S * V),
                table_1, table_2, table_3, table_4, table_5, table_6)


# speedup vs baseline: 1.4992x; 1.0078x over previous
import functools

import jax
import jax.numpy as jnp
from jax import lax
from jax.experimental import pallas as pl
from jax.experimental.pallas import tpu as pltpu
from jax.experimental.pallas import tpu_sc as plsc

V = 8
S = 7
D = 64
B = 16384

NC = 2
NS = 16
L = 16
NW = NC * NS
BPW = B // NW
CHUNK = 128
NCH = BPW // CHUNK
GPC = CHUNK // L


def _body(in_hbm, t1, t2, t3, t4, t5, t6, out_ref,
          in_v, idx_v, gbuf, zbuf, sem, wsem):
    wid = lax.axis_index("s") * NC + lax.axis_index("c")
    base = wid * BPW
    tabs = [t1, t2, t3, t4, t5, t6]

    pltpu.sync_copy(in_hbm.at[pl.ds(0, 6), :, pl.ds(base, BPW)], in_v)

    iota = jax.lax.iota(jnp.int32, L)

    def zero_body(r, _):
        for c in range(D // L):
            zbuf[r, pl.ds(c * L, L)] = jnp.zeros((L,), jnp.float32)
        return _
    lax.fori_loop(0, CHUNK, zero_body, None)

    for j in range(NCH):
        def amax_body(gg, _):
            off = j * CHUNK + gg * L
            e = jnp.zeros((L,), jnp.int32)
            for s in range(S - 1):
                m = in_v[s, 0, pl.ds(off, L)]
                a = jnp.zeros((L,), jnp.int32)
                for v in range(1, V):
                    val = in_v[s, v, pl.ds(off, L)]
                    gt = val > m
                    m = jnp.where(gt, val, m)
                    a = jnp.where(gt, jnp.full((L,), v, jnp.int32), a)
                e = e + a * (V ** s)
                idx_v[s, j, pl.ds(gg * L, L)] = e
            return _
        lax.fori_loop(0, GPC, amax_body, None)

        copies = []
        for d in range(6):
            copies.append(pltpu.make_async_copy(
                tabs[d].at[idx_v.at[d, j]], gbuf.at[d], sem))
        for c in copies:
            c.start()
        for c in copies:
            c.wait()

        rbase = base + j * CHUNK
        writes = [pltpu.make_async_copy(
            zbuf, out_ref.at[pl.ds(rbase, CHUNK), pl.ds(0, D)], wsem)]
        for d in range(6):
            writes.append(pltpu.make_async_copy(
                gbuf.at[d],
                out_ref.at[pl.ds(rbase, CHUNK), pl.ds((d + 1) * D, D)],
                wsem))
        for w in writes:
            w.start()
        for w in writes:
            w.wait()


_kern = functools.partial(
    pl.kernel,
    out_type=jax.ShapeDtypeStruct((B, S * D), jnp.float32),
    mesh=plsc.VectorSubcoreMesh(core_axis_name="c", subcore_axis_name="s"),
    compiler_params=pltpu.CompilerParams(use_tc_tiling_on_sc=False,
                                         needs_layout_passes=False),
    scratch_types=[
        pltpu.VMEM((6, V, BPW), jnp.float32),
        pltpu.VMEM((6, NCH, CHUNK), jnp.int32),
        pltpu.VMEM((6, CHUNK, D), jnp.float32),
        pltpu.VMEM((CHUNK, D), jnp.float32),
        pltpu.SemaphoreType.DMA,
        pltpu.SemaphoreType.DMA,
    ],
)(_body)


@jax.jit
def _run(inputs2d, t1, t2, t3, t4, t5, t6):
    return _kern(inputs2d, t1, t2, t3, t4, t5, t6).reshape(B, S, D)


def kernel(inputs, table_1, table_2, table_3, table_4, table_5, table_6):
    return _run(inputs.transpose(1, 2, 0),
                table_1, table_2, table_3, table_4, table_5, table_6)
